# baseline (speedup 1.0000x reference)
import functools

import jax
import jax.numpy as jnp
from jax import lax
from jax.experimental import pallas as pl
from jax.experimental.pallas import tpu as pltpu
from jax.experimental.pallas import tpu_sc as plsc

D = 64
S = 200
NC = 2
NS = 16
NW = NC * NS
BB = 128
L = 16
PR = 2 * D
PRS = PR + 8


def _body(x_ref, tok_ref, pos_ref, out_ref,
          idx_v, pos_v,
          gbuf0, gbuf1, sbuf0, sbuf1, idxd0, idxd1,
          gsem0, gsem1, ssem0, ssem1):
  w = lax.axis_index("s") * NC + lax.axis_index("c")
  b0 = w * BB

  pltpu.sync_copy(x_ref.at[:, pl.ds(b0, BB)], idx_v)
  pltpu.sync_copy(pos_ref, pos_v)

  gbufs = (gbuf0, gbuf1)
  sbufs = (sbuf0, sbuf1)
  idxds = (idxd0, idxd1)
  gsems = (gsem0, gsem1)
  ssems = (ssem0, ssem1)

  dvecs = [lax.iota(jnp.int32, L) + j * L for j in range(D // L)]
  rsplats = [jnp.full((L,), r, jnp.int32) for r in range(BB)]

  def build_idx(s, idxd):
    for j in range(BB // L):
      t = idx_v[s, pl.ds(j * L, L)]
      idxd[pl.ds(j * L, L)] = lax.shift_left(
          lax.shift_right_logical(t, 8), 7) | (t & 127)

  def step(s, gb, sb, idxd, gsem, ssem):
    pltpu.make_async_copy(tok_ref.at[idxd], gb.at[:, pl.ds(0, PR)], gsem).wait()

    hvs = [lax.shift_left(
               lax.shift_right_logical(idx_v[s, pl.ds(j * L, L)], 7) & 1, 6)
           for j in range(BB // L)]

    p = [pos_v[pl.ds(s * D + j * L, L)] for j in range(D // L)]

    @pl.when(s >= 2)
    def _():
      pltpu.make_async_copy(
          sb, out_ref.at[0, :, pl.ds(b0, BB)], ssem).wait()

    for bc in range(BB // L):
      hv = hvs[bc]
      for i in range(L):
        r = bc * L + i
        h = hv[i]
        rs = rsplats[r]
        for j in range(D // L):
          val = gb[r, pl.ds(h + j * L, L)] + p[j]
          plsc.store_scatter(sb, [dvecs[j], rs], val)

    pltpu.async_copy(sb, out_ref.at[s, :, pl.ds(b0, BB)], ssem)

    @pl.when(s + 2 < S)
    def _():
      build_idx(s + 2, idxd)
      pltpu.async_copy(tok_ref.at[idxd], gb.at[:, pl.ds(0, PR)], gsem)

  for b in range(2):
    build_idx(b, idxds[b])
    pltpu.async_copy(tok_ref.at[idxds[b]], gbufs[b].at[:, pl.ds(0, PR)], gsems[b])

  @pl.loop(0, S, step=2)
  def _(s0):
    for b in range(2):
      step(s0 + b, gbufs[b], sbufs[b], idxds[b], gsems[b], ssems[b])

  for b in range(2):
    pltpu.make_async_copy(
        sbufs[b], out_ref.at[0, :, pl.ds(b0, BB)], ssems[b]).wait()


PC = 4096


def _pack_body(src_ref, out_ref):
  a = src_ref[...]
  parts = []
  for p in range(PC // 256):
    parts.append(jnp.concatenate(
        [a[:, 256 * p:256 * p + 128].T, a[:, 256 * p + 128:256 * p + 256].T],
        axis=1))
  out_ref[...] = jnp.concatenate(parts, axis=0)


def _pack_table(tokT):
  v = tokT.shape[1]
  grid = (v + PC - 1) // PC
  return pl.pallas_call(
      _pack_body,
      grid=(grid,),
      in_specs=[pl.BlockSpec((D, PC), lambda i: (0, i))],
      out_specs=pl.BlockSpec((PC // 2, PR), lambda i: (i, 0)),
      out_shape=jax.ShapeDtypeStruct((grid * PC // 2, PR), jnp.float32),
  )(tokT)


@jax.jit
def kernel(x, token_table, pos_table):
  bs, seq_len = x.shape
  xT = x.T
  tok2 = _pack_table(token_table.T)
  posf = pos_table.reshape(-1)

  fn = pl.kernel(
      _body,
      out_type=jax.ShapeDtypeStruct((S, D, bs), jnp.float32),
      mesh=plsc.VectorSubcoreMesh(core_axis_name="c", subcore_axis_name="s"),
      compiler_params=pltpu.CompilerParams(needs_layout_passes=False),
      scratch_types=[
          pltpu.VMEM((S, BB), jnp.int32),
          pltpu.VMEM((S * D,), jnp.float32),
          pltpu.VMEM((BB, PRS), jnp.float32),
          pltpu.VMEM((BB, PRS), jnp.float32),
          pltpu.VMEM((D, BB), jnp.float32),
          pltpu.VMEM((D, BB), jnp.float32),
          pltpu.VMEM((BB,), jnp.int32),
          pltpu.VMEM((BB,), jnp.int32),
          pltpu.SemaphoreType.DMA,
          pltpu.SemaphoreType.DMA,
          pltpu.SemaphoreType.DMA,
          pltpu.SemaphoreType.DMA,
      ],
  )
  out3 = fn(xT, tok2, posf)
  return out3.transpose(2, 0, 1)


# speedup vs baseline: 1.6014x; 1.6014x over previous
import functools

import jax
import jax.numpy as jnp
from jax import lax
from jax.experimental import pallas as pl
from jax.experimental.pallas import tpu as pltpu
from jax.experimental.pallas import tpu_sc as plsc

D = 64
S = 200
NC = 2
NS = 16
NW = NC * NS
BB = 128
L = 16
PR = 2 * D
PRS = PR + 8


def _body(x_ref, tok_ref, pos_ref, out_ref,
          idx_v, pos_v,
          gbuf0, gbuf1, sbuf0, sbuf1, idxd0, idxd1,
          gsem0, gsem1, ssem0, ssem1):
  w = lax.axis_index("s") * NC + lax.axis_index("c")
  b0 = w * BB

  pltpu.sync_copy(x_ref.at[:, pl.ds(b0, BB)], idx_v)
  pltpu.sync_copy(pos_ref, pos_v)

  gbufs = (gbuf0, gbuf1)
  sbufs = (sbuf0, sbuf1)
  idxds = (idxd0, idxd1)
  gsems = (gsem0, gsem1)
  ssems = (ssem0, ssem1)

  rows = [lax.iota(jnp.int32, L) + bc * L for bc in range(BB // L)]

  def build_idx(s, idxd):
    for j in range(BB // L):
      t = idx_v[s, pl.ds(j * L, L)]
      idxd[pl.ds(j * L, L)] = lax.shift_left(
          lax.shift_right_logical(t, 8), 7) | (t & 127)

  def step(s, gb, sb, idxd, gsem, ssem):
    pltpu.make_async_copy(tok_ref.at[idxd], gb.at[:, pl.ds(0, PR)], gsem).wait()

    hvs = [lax.shift_left(
               lax.shift_right_logical(idx_v[s, pl.ds(j * L, L)], 7) & 1, 6)
           for j in range(BB // L)]

    @pl.when(s >= 2)
    def _():
      pltpu.make_async_copy(
          sb, out_ref.at[0, :, pl.ds(b0, BB)], ssem).wait()

    sD = s * D

    @plsc.parallel_loop(0, D, unroll=8)
    def _(d):
      pd = plsc.load_gather(pos_v, [jnp.broadcast_to(sD + d, (L,))])
      for bc in range(BB // L):
        val = plsc.load_gather(gb, [rows[bc], hvs[bc] + d])
        sb[d, pl.ds(bc * L, L)] = val + pd

    pltpu.async_copy(sb, out_ref.at[s, :, pl.ds(b0, BB)], ssem)

    @pl.when(s + 2 < S)
    def _():
      build_idx(s + 2, idxd)
      pltpu.async_copy(tok_ref.at[idxd], gb.at[:, pl.ds(0, PR)], gsem)

  for b in range(2):
    build_idx(b, idxds[b])
    pltpu.async_copy(tok_ref.at[idxds[b]], gbufs[b].at[:, pl.ds(0, PR)], gsems[b])

  @pl.loop(0, S, step=2)
  def _(s0):
    for b in range(2):
      step(s0 + b, gbufs[b], sbufs[b], idxds[b], gsems[b], ssems[b])

  for b in range(2):
    pltpu.make_async_copy(
        sbufs[b], out_ref.at[0, :, pl.ds(b0, BB)], ssems[b]).wait()


PC = 8192


def _pack_body(src_ref, out_ref):
  a = src_ref[...]
  parts = []
  for p in range(PC // 256):
    parts.append(jnp.concatenate(
        [a[:, 256 * p:256 * p + 128].T, a[:, 256 * p + 128:256 * p + 256].T],
        axis=1))
  out_ref[...] = jnp.concatenate(parts, axis=0)


def _pack_table(tokT):
  v = tokT.shape[1]
  grid = (v + PC - 1) // PC
  return pl.pallas_call(
      _pack_body,
      grid=(grid,),
      in_specs=[pl.BlockSpec((D, PC), lambda i: (0, i))],
      out_specs=pl.BlockSpec((PC // 2, PR), lambda i: (i, 0)),
      out_shape=jax.ShapeDtypeStruct((grid * PC // 2, PR), jnp.float32),
  )(tokT)


@jax.jit
def kernel(x, token_table, pos_table):
  bs, seq_len = x.shape
  xT = x.T
  tok2 = _pack_table(token_table.T)
  posf = pos_table.reshape(-1)

  fn = pl.kernel(
      _body,
      out_type=jax.ShapeDtypeStruct((S, D, bs), jnp.float32),
      mesh=plsc.VectorSubcoreMesh(core_axis_name="c", subcore_axis_name="s"),
      compiler_params=pltpu.CompilerParams(needs_layout_passes=False),
      scratch_types=[
          pltpu.VMEM((S, BB), jnp.int32),
          pltpu.VMEM((S * D,), jnp.float32),
          pltpu.VMEM((BB, PRS), jnp.float32),
          pltpu.VMEM((BB, PRS), jnp.float32),
          pltpu.VMEM((D, BB), jnp.float32),
          pltpu.VMEM((D, BB), jnp.float32),
          pltpu.VMEM((BB,), jnp.int32),
          pltpu.VMEM((BB,), jnp.int32),
          pltpu.SemaphoreType.DMA,
          pltpu.SemaphoreType.DMA,
          pltpu.SemaphoreType.DMA,
          pltpu.SemaphoreType.DMA,
      ],
  )
  out3 = fn(xT, tok2, posf)
  return out3.transpose(2, 0, 1)


# speedup vs baseline: 1.6849x; 1.0521x over previous
import functools

import jax
import jax.numpy as jnp
from jax import lax
from jax.experimental import pallas as pl
from jax.experimental.pallas import tpu as pltpu
from jax.experimental.pallas import tpu_sc as plsc

D = 64
S = 200
NC = 2
NS = 16
NW = NC * NS
BB = 128
L = 16
PR = 2 * D
PRS = PR


def _body(x_ref, tok_ref, pos_ref, out_ref,
          idx_v, pos_v,
          gbuf0, gbuf1, sbuf0, sbuf1, idxd0, idxd1,
          gsem0, gsem1, ssem0, ssem1):
  w = lax.axis_index("s") * NC + lax.axis_index("c")
  b0 = w * BB

  pltpu.sync_copy(x_ref.at[:, pl.ds(b0, BB)], idx_v)
  pltpu.sync_copy(pos_ref, pos_v)

  gbufs = (gbuf0, gbuf1)
  sbufs = (sbuf0, sbuf1)
  idxds = (idxd0, idxd1)
  gsems = (gsem0, gsem1)
  ssems = (ssem0, ssem1)

  rows = [lax.iota(jnp.int32, L) + bc * L for bc in range(BB // L)]

  def build_idx(s, idxd):
    for j in range(BB // L):
      t = idx_v[s, pl.ds(j * L, L)]
      idxd[pl.ds(j * L, L)] = lax.shift_left(
          lax.shift_right_logical(t, 8), 7) | (t & 127)

  def step(s, gb, sb, idxd, gsem, ssem):
    pltpu.make_async_copy(tok_ref.at[idxd], gb.at[:, pl.ds(0, PR)], gsem).wait()

    hvs = [lax.shift_left(
               lax.shift_right_logical(idx_v[s, pl.ds(j * L, L)], 7) & 1, 6)
           for j in range(BB // L)]

    @pl.when(s >= 2)
    def _():
      pltpu.make_async_copy(
          sb, out_ref.at[0, :, pl.ds(b0, BB)], ssem).wait()

    sD = s * D

    @plsc.parallel_loop(0, D, unroll=8)
    def _(d):
      pd = plsc.load_gather(pos_v, [jnp.broadcast_to(sD + d, (L,))])
      for bc in range(BB // L):
        val = plsc.load_gather(gb, [rows[bc], hvs[bc] + d])
        sb[d, pl.ds(bc * L, L)] = val + pd

    pltpu.async_copy(sb, out_ref.at[s, :, pl.ds(b0, BB)], ssem)

    @pl.when(s + 2 < S)
    def _():
      build_idx(s + 2, idxd)
      pltpu.async_copy(tok_ref.at[idxd], gb.at[:, pl.ds(0, PR)], gsem)

  for b in range(2):
    build_idx(b, idxds[b])
    pltpu.async_copy(tok_ref.at[idxds[b]], gbufs[b].at[:, pl.ds(0, PR)], gsems[b])

  @pl.loop(0, S, step=2)
  def _(s0):
    for b in range(2):
      step(s0 + b, gbufs[b], sbufs[b], idxds[b], gsems[b], ssems[b])

  for b in range(2):
    pltpu.make_async_copy(
        sbufs[b], out_ref.at[0, :, pl.ds(b0, BB)], ssems[b]).wait()


PC = 16384


def _pack_body(src_ref, out_ref):
  a = src_ref[...]
  parts = []
  for p in range(PC // 256):
    parts.append(jnp.concatenate(
        [a[:, 256 * p:256 * p + 128].T, a[:, 256 * p + 128:256 * p + 256].T],
        axis=1))
  out_ref[...] = jnp.concatenate(parts, axis=0)


def _pack_table(tokT):
  v = tokT.shape[1]
  grid = (v + PC - 1) // PC
  return pl.pallas_call(
      _pack_body,
      grid=(grid,),
      in_specs=[pl.BlockSpec((D, PC), lambda i: (0, i))],
      out_specs=pl.BlockSpec((PC // 2, PR), lambda i: (i, 0)),
      out_shape=jax.ShapeDtypeStruct((grid * PC // 2, PR), jnp.float32),
  )(tokT)


@jax.jit
def kernel(x, token_table, pos_table):
  bs, seq_len = x.shape
  xT = x.T
  tok2 = _pack_table(token_table.T)
  posf = pos_table.reshape(-1)

  fn = pl.kernel(
      _body,
      out_type=jax.ShapeDtypeStruct((S, D, bs), jnp.float32),
      mesh=plsc.VectorSubcoreMesh(core_axis_name="c", subcore_axis_name="s"),
      compiler_params=pltpu.CompilerParams(needs_layout_passes=False),
      scratch_types=[
          pltpu.VMEM((S, BB), jnp.int32),
          pltpu.VMEM((S * D,), jnp.float32),
          pltpu.VMEM((BB, PRS), jnp.float32),
          pltpu.VMEM((BB, PRS), jnp.float32),
          pltpu.VMEM((D, BB), jnp.float32),
          pltpu.VMEM((D, BB), jnp.float32),
          pltpu.VMEM((BB,), jnp.int32),
          pltpu.VMEM((BB,), jnp.int32),
          pltpu.SemaphoreType.DMA,
          pltpu.SemaphoreType.DMA,
          pltpu.SemaphoreType.DMA,
          pltpu.SemaphoreType.DMA,
      ],
  )
  out3 = fn(xT, tok2, posf)
  return out3.transpose(2, 0, 1)
